# Initial kernel scaffold; baseline (speedup 1.0000x reference)
import jax
import jax.numpy as jnp
from jax.experimental import pallas as pl


def kernel(queries, keys, values, padding_mask, Wq, bq, Wk, bk, Wv, bv, Wf, bf, qln_w, qln_b, fln_w, fln_b):
    raise NotImplementedError("write your pallas kernel here")


# speedup vs baseline: 5.4965x; 5.4965x over previous
import functools
import math

import numpy as np
import jax
import jax.numpy as jnp
from jax import lax
from jax.experimental import pallas as pl
from jax.experimental.pallas import tpu as pltpu
from jax.experimental.pallas import tpu_sc as plsc

B, L, D, H = 1, 2048, 1024, 16
DH = D // H
ALPHA = 5
U = min(ALPHA * int(np.ceil(np.log(L))), L)
BLK = 256
NBLK = L // BLK
HIGHEST = lax.Precision.HIGHEST


@functools.lru_cache(maxsize=None)
def _sample_count_matrix():
    with jax.ensure_compile_time_eval():
        idx = np.asarray(jax.random.randint(jax.random.key(42), (L, U), 0, L))
    cnt = np.zeros((L, L), np.float32)
    np.add.at(cnt, (np.arange(L)[:, None], idx), 1.0)
    return cnt


def _proj_body(xq_ref, xk_ref, xv_ref, padc_ref,
               wq_ref, bq_ref, wk_ref, bk_ref, wv_ref, bv_ref,
               qlw_ref, qlb_ref,
               q_out, k_out, v_out, va_out,
               vcarry, pcarry):
    i = pl.program_id(0)

    @pl.when(i == 0)
    def _():
        vcarry[...] = jnp.zeros_like(vcarry)
        pcarry[0] = 0.0

    x = xq_ref[...]
    u = jnp.mean(x, axis=-1, keepdims=True)
    s = jnp.mean((x - u) ** 2, axis=-1, keepdims=True)
    qn = qlw_ref[...] * (x - u) / jnp.sqrt(s + 1e-8) + qlb_ref[...]
    q_out[...] = lax.dot_general(qn, wq_ref[...], (((1,), (1,)), ((), ()))) + bq_ref[...]
    k_out[...] = lax.dot_general(xk_ref[...], wk_ref[...], (((1,), (1,)), ((), ()))) + bk_ref[...]
    v = lax.dot_general(xv_ref[...], wv_ref[...], (((1,), (1,)), ((), ()))) + bv_ref[...]
    v_out[...] = v

    r = lax.broadcasted_iota(jnp.int32, (BLK, BLK), 0)
    c = lax.broadcasted_iota(jnp.int32, (BLK, BLK), 1)
    tri = (r >= c).astype(jnp.float32)
    csum = lax.dot_general(tri, v, (((1,), (0,)), ((), ())), precision=HIGHEST) + vcarry[...]
    pc = lax.dot_general(tri, padc_ref[...], (((1,), (0,)), ((), ())), precision=HIGHEST) + pcarry[0]
    va_out[...] = csum / (pc + 1e-12)
    vcarry[...] = csum[BLK - 1:BLK, :]
    pcarry[0] = pc[BLK - 1, 0]


def _stage1(xq, xk, xv, padcol, Wq, bq, Wk, bk, Wv, bv, qln_w, qln_b):
    full = pl.BlockSpec((D, D), lambda i: (0, 0))
    row = pl.BlockSpec((1, D), lambda i: (0, 0))
    blk = pl.BlockSpec((BLK, D), lambda i: (i, 0))
    return pl.pallas_call(
        _proj_body,
        grid=(NBLK,),
        in_specs=[blk, blk, blk,
                  pl.BlockSpec((BLK, 1), lambda i: (i, 0)),
                  full, row, full, row, full, row, row, row],
        out_specs=[blk, blk, blk, blk],
        out_shape=[jax.ShapeDtypeStruct((L, D), jnp.float32)] * 4,
        scratch_shapes=[pltpu.VMEM((1, D), jnp.float32),
                        pltpu.SMEM((1,), jnp.float32)],
    )(xq, xk, xv, padcol, Wq, bq.reshape(1, D), Wk, bk.reshape(1, D),
      Wv, bv.reshape(1, D), qln_w.reshape(1, D), qln_b.reshape(1, D))


def _m_body(q_ref, k_ref, cnt_ref, m_out):
    s = lax.dot_general(q_ref[0], k_ref[0], (((1,), (1,)), ((), ())))
    cnt = cnt_ref[...]
    masked = jnp.where(cnt > 0.0, s, -1e30)
    mx = jnp.max(masked, axis=-1, keepdims=True)
    sm = jnp.sum(s * cnt, axis=-1, keepdims=True)
    m_out[...] = (mx - sm * (1.0 / L)).reshape(1, 1, BLK, 1)


def _stage2(q3, k3, cnt):
    m4 = pl.pallas_call(
        _m_body,
        grid=(NBLK, H),
        in_specs=[pl.BlockSpec((1, BLK, DH), lambda i, h: (h, i, 0)),
                  pl.BlockSpec((1, L, DH), lambda i, h: (h, 0, 0)),
                  pl.BlockSpec((BLK, L), lambda i, h: (i, 0))],
        out_specs=pl.BlockSpec((1, 1, BLK, 1), lambda i, h: (i, h, 0, 0)),
        out_shape=jax.ShapeDtypeStruct((NBLK, H, BLK, 1), jnp.float32),
    )(q3, k3, cnt)
    return m4.reshape(NBLK, H, BLK).transpose(1, 0, 2).reshape(H, L)


def _topk_sc(m):
    mesh = plsc.VectorSubcoreMesh(core_axis_name="c", subcore_axis_name="s")

    @functools.partial(
        pl.kernel,
        out_type=jax.ShapeDtypeStruct((H, U), jnp.int32),
        mesh=mesh,
        scratch_types=[pltpu.VMEM((L,), jnp.float32),
                       pltpu.VMEM((U,), jnp.int32)],
        compiler_params=pltpu.CompilerParams(needs_layout_passes=False),
    )
    def topk_kernel(m_hbm, out_hbm, m_v, out_v):
        wid = lax.axis_index("s") * 2 + lax.axis_index("c")

        @pl.when(wid < H)
        def _():
            pltpu.sync_copy(m_hbm.at[wid], m_v)
            lanes = lax.iota(jnp.int32, 16)
            lane0 = lanes == 0

            def outer(t, carry):
                def inner(j, bc):
                    best, bidx = bc
                    v = m_v[pl.ds(j * 16, 16)]
                    upd = v > best
                    return (jnp.where(upd, v, best),
                            jnp.where(upd, j * 16 + lanes, bidx))

                best, bidx = lax.fori_loop(
                    0, L // 16, inner,
                    (jnp.full((16,), -3e38, jnp.float32),
                     jnp.zeros((16,), jnp.int32)))
                _, sv = plsc.sort_key_val(best, bidx, descending=True)
                plsc.store_scatter(out_v, [jnp.full((16,), t, jnp.int32)],
                                   sv, mask=lane0)
                plsc.store_scatter(m_v, [sv],
                                   jnp.full((16,), -3e38, jnp.float32),
                                   mask=lane0)
                return carry

            lax.fori_loop(0, U, outer, 0)
            pltpu.sync_copy(out_v, out_hbm.at[wid])

    return topk_kernel(m)


def _attn_body(top_ref, pad_ref, q_ref, k_ref, v_ref, va_ref, o_ref,
               qr_scr, msk_scr):
    h = pl.program_id(0)
    o_ref[...] = va_ref[...]
    ii = lax.broadcasted_iota(jnp.int32, (1, L), 1)
    for i in range(U):
        idx = top_ref[h, i]
        qr_scr[i:i + 1, :] = q_ref[0, pl.ds(idx, 1), :]
        p = pad_ref[0, idx]
        mrow = jnp.logical_and(ii <= idx, p != 0.0)
        msk_scr[i:i + 1, :] = mrow.astype(jnp.float32)
    scores = lax.dot_general(qr_scr[...], k_ref[0],
                             (((1,), (1,)), ((), ()))) * (1.0 / math.sqrt(D))
    scores = jnp.where(msk_scr[...] > 0.0, scores, -100000.0)
    mx = jnp.max(scores, axis=-1, keepdims=True)
    e = jnp.exp(scores - mx)
    attn = e / jnp.sum(e, axis=-1, keepdims=True)
    ctx = lax.dot_general(attn, v_ref[0], (((1,), (0,)), ((), ())))
    for i in range(U):
        idx = top_ref[h, i]
        o_ref[0, pl.ds(idx, 1), :] = ctx[i:i + 1, :]


def _stage4(top, padrow, q3, k3, v3, va3):
    col = pl.BlockSpec((1, L, DH), lambda h: (h, 0, 0))
    return pl.pallas_call(
        _attn_body,
        grid=(H,),
        in_specs=[pl.BlockSpec(memory_space=pltpu.SMEM),
                  pl.BlockSpec(memory_space=pltpu.SMEM),
                  col, col, col, col],
        out_specs=col,
        out_shape=jax.ShapeDtypeStruct((H, L, DH), jnp.float32),
        scratch_shapes=[pltpu.VMEM((U, DH), jnp.float32),
                        pltpu.VMEM((U, L), jnp.float32)],
    )(top, padrow, q3, k3, v3, va3)


def _final_body(x_ref, res_ref, wf_ref, bf_ref, flw_ref, flb_ref, o_ref):
    y = lax.dot_general(x_ref[...], wf_ref[...],
                        (((1,), (1,)), ((), ()))) + bf_ref[...] + res_ref[...]
    u = jnp.mean(y, axis=-1, keepdims=True)
    s = jnp.mean((y - u) ** 2, axis=-1, keepdims=True)
    o_ref[...] = flw_ref[...] * (y - u) / jnp.sqrt(s + 1e-8) + flb_ref[...]


def _stage5(ctxfull, xq, Wf, bf, fln_w, fln_b):
    blk = pl.BlockSpec((BLK, D), lambda i: (i, 0))
    full = pl.BlockSpec((D, D), lambda i: (0, 0))
    row = pl.BlockSpec((1, D), lambda i: (0, 0))
    return pl.pallas_call(
        _final_body,
        grid=(NBLK,),
        in_specs=[blk, blk, full, row, row, row],
        out_specs=blk,
        out_shape=jax.ShapeDtypeStruct((L, D), jnp.float32),
    )(ctxfull, xq, Wf, bf.reshape(1, D), fln_w.reshape(1, D),
      fln_b.reshape(1, D))


def kernel(queries, keys, values, padding_mask, Wq, bq, Wk, bk, Wv, bv,
           Wf, bf, qln_w, qln_b, fln_w, fln_b):
    xq = queries.reshape(L, D)
    xk = keys.reshape(L, D)
    xv = values.reshape(L, D)
    padcol = padding_mask.reshape(L, 1)
    padrow = padding_mask.reshape(1, L)

    q, k, v, va = _stage1(xq, xk, xv, padcol,
                          Wq, bq, Wk, bk, Wv, bv, qln_w, qln_b)
    q3 = q.reshape(L, H, DH).transpose(1, 0, 2)
    k3 = k.reshape(L, H, DH).transpose(1, 0, 2)
    v3 = v.reshape(L, H, DH).transpose(1, 0, 2)
    va3 = va.reshape(L, H, DH).transpose(1, 0, 2)
    cnt = jnp.asarray(_sample_count_matrix())
    m = _stage2(q3, k3, cnt)
    top = _topk_sc(m)
    ctx3 = _stage4(top, padrow, q3, k3, v3, va3)
    ctxfull = ctx3.transpose(1, 0, 2).reshape(L, D)
    out = _stage5(ctxfull, xq, Wf, bf, fln_w, fln_b)
    return out.reshape(B, L, D)
